# baseline (speedup 1.0000x reference)
import jax
import jax.numpy as jnp
from jax import lax
from jax.experimental import pallas as pl
from jax.experimental.pallas import tpu as pltpu
from jax.experimental.pallas import tpu_sc as plsc

N_NODES = 10000
N_PAD = 10240
D = 128
N_EDGES = 320000

NC = 2
NS = 16
NW = NC * NS
CHUNK = 128
STEPS = 80
N_CHUNKS = NW * STEPS
E_PAD = N_CHUNKS * CHUNK
DEG_STEPS = N_CHUNKS // NW
ROWS_PER_TILE = N_PAD // NS
DEG_W = 16
IB = 8
NG = STEPS // IB
DEG_GRP = 8


def _deg_sc_body(dst_hbm, cnt_hbm, didx_all, ones_v, zbuf, deg_sh, dsem):
    c = lax.axis_index("c")
    s = lax.axis_index("s")
    wid = c * NS + s
    rlo = s * ROWS_PER_TILE

    one16 = jnp.full((16,), 1.0, dtype=jnp.float32)
    zero16 = jnp.zeros((16,), dtype=jnp.float32)
    def fill(i, _):
        ones_v[i, pl.ds(0, 16)] = one16
        zbuf[i, pl.ds(0, 16)] = zero16
        return 0
    lax.fori_loop(0, CHUNK, fill, 0)

    pltpu.sync_copy(dst_hbm.at[pl.ds(wid * DEG_STEPS, DEG_STEPS)], didx_all)
    for k in range(ROWS_PER_TILE // CHUNK):
        pltpu.sync_copy(zbuf, deg_sh.at[pl.ds(rlo + k * CHUNK, CHUNK)])
    plsc.subcore_barrier()

    def group(g, _):
        for j in range(DEG_GRP):
            pltpu.async_copy(ones_v, deg_sh.at[didx_all.at[g * DEG_GRP + j]],
                             dsem, add=True)
        for j in range(DEG_GRP):
            pltpu.make_async_copy(ones_v, deg_sh.at[didx_all.at[0]],
                                  dsem).wait()
        return 0
    lax.fori_loop(0, DEG_STEPS // DEG_GRP, group, 0)

    plsc.subcore_barrier()
    pltpu.sync_copy(deg_sh.at[pl.ds(rlo, ROWS_PER_TILE)],
                    cnt_hbm.at[c, pl.ds(rlo, ROWS_PER_TILE)])


def _scatter_sc_body(y_hbm, src_hbm, dst_hbm, z_hbm,
                     sidx3, didx3, rb0, rb1, z_sh,
                     isem0, isem1, gsem0, gsem1, ssem0, ssem1):
    c = lax.axis_index("c")
    s = lax.axis_index("s")
    wid = c * NS + s
    rlo = s * ROWS_PER_TILE
    rows = [rb0, rb1]
    isem = [isem0, isem1]
    gsem = [gsem0, gsem1]
    ssem = [ssem0, ssem1]
    cbase = wid * STEPS

    zero16 = jnp.zeros((16,), dtype=jnp.float32)
    def fill(i, _):
        for j in range(D // 16):
            rb0[i, pl.ds(j * 16, 16)] = zero16
        return 0
    lax.fori_loop(0, CHUNK, fill, 0)
    for k in range(ROWS_PER_TILE // CHUNK):
        pltpu.sync_copy(rb0, z_sh.at[pl.ds(rlo + k * CHUNK, CHUNK)])
    plsc.subcore_barrier()

    def iload(gi, p):
        pltpu.async_copy(src_hbm.at[pl.ds(cbase + gi * IB, IB)],
                         sidx3.at[p], isem[p])
        pltpu.async_copy(dst_hbm.at[pl.ds(cbase + gi * IB, IB)],
                         didx3.at[p], isem[p])
    def iwait(p):
        pltpu.make_async_copy(src_hbm.at[pl.ds(cbase, IB)], sidx3.at[p],
                              isem[p]).wait()
        pltpu.make_async_copy(dst_hbm.at[pl.ds(cbase, IB)], didx3.at[p],
                              isem[p]).wait()
    def gstart(p, k, b):
        pltpu.async_copy(y_hbm.at[sidx3.at[p, k]], rows[b], gsem[b])
    def gwait(b):
        pltpu.make_async_copy(y_hbm.at[sidx3.at[0, 0]], rows[b],
                              gsem[b]).wait()
    def sstart(p, k, b):
        pltpu.async_copy(rows[b], z_sh.at[didx3.at[p, k]], ssem[b], add=True)
    def swait(b):
        pltpu.make_async_copy(rows[b], z_sh.at[didx3.at[0, 0]],
                              ssem[b]).wait()

    def group(gi, p):
        iwait(p)
        d = pltpu.async_copy(y_hbm.at[sidx3.at[p, 0]], rows[0], gsem[0])
        sd = [None, None]
        for k in range(IB):
            b = k % 2
            d.wait()
            if sd[b] is not None:
                sd[b].wait()
            sd[b] = pltpu.async_copy(rows[b], z_sh.at[didx3.at[p, k]],
                                     ssem[b], add=True)
            if k + 1 < IB:
                if sd[1 - b] is not None:
                    sd[1 - b].wait()
                    sd[1 - b] = None
                d = pltpu.async_copy(y_hbm.at[sidx3.at[p, k + 1]],
                                     rows[1 - b], gsem[1 - b])
        sd[0].wait()
        sd[1].wait()
        @pl.when(gi < NG - 2)
        def _():
            iload(gi + 2, p)

    iload(0, 0)
    iload(1, 1)
    def outer(t, _):
        group(2 * t, 0)
        group(2 * t + 1, 1)
        return 0
    lax.fori_loop(0, NG // 2, outer, 0)

    plsc.subcore_barrier()
    pltpu.sync_copy(z_sh.at[pl.ds(rlo, ROWS_PER_TILE)],
                    z_hbm.at[c, pl.ds(rlo, ROWS_PER_TILE)])


def _scale_tc_body(x_ref, cnt_ref, y_ref, s_ref):
    cnt = cnt_ref[0, 0:N_NODES, 0:1] + cnt_ref[1, 0:N_NODES, 0:1]
    s = lax.rsqrt(cnt + 1.0)
    s_ref[...] = s
    y_ref[0:N_NODES, :] = x_ref[...] * s
    y_ref[N_NODES:N_PAD, :] = jnp.zeros((N_PAD - N_NODES, D), jnp.float32)


def _matmul_tc_body(z_ref, y_ref, s_ref, wm_ref, bm_ref, wl_ref, bl_ref,
                    mu_ref, ls_ref):
    agg = (z_ref[0, 0:N_NODES, :] + z_ref[1, 0:N_NODES, :]
           + y_ref[0:N_NODES, :]) * s_ref[...]
    mu_ref[...] = jnp.dot(agg, wm_ref[...],
                          preferred_element_type=jnp.float32,
                          precision=lax.Precision.HIGHEST) + bm_ref[...]
    ls_ref[...] = jnp.dot(agg, wl_ref[...],
                          preferred_element_type=jnp.float32,
                          precision=lax.Precision.HIGHEST) + bl_ref[...]


_SC_MESH = plsc.VectorSubcoreMesh(core_axis_name="c", subcore_axis_name="s")

_deg_call = pl.kernel(
    _deg_sc_body,
    out_type=jax.ShapeDtypeStruct((NC, N_PAD, DEG_W), jnp.float32),
    mesh=_SC_MESH,
    scratch_types=[
        pltpu.VMEM((DEG_STEPS, CHUNK), jnp.int32),
        pltpu.VMEM((CHUNK, DEG_W), jnp.float32),
        pltpu.VMEM((CHUNK, DEG_W), jnp.float32),
        pltpu.VMEM_SHARED((N_PAD, DEG_W), jnp.float32),
        pltpu.SemaphoreType.DMA,
    ],
)

_scatter_call = pl.kernel(
    _scatter_sc_body,
    out_type=jax.ShapeDtypeStruct((NC, N_PAD, D), jnp.float32),
    mesh=_SC_MESH,
    scratch_types=[
        pltpu.VMEM((2, IB, CHUNK), jnp.int32),
        pltpu.VMEM((2, IB, CHUNK), jnp.int32),
        pltpu.VMEM((CHUNK, D), jnp.float32),
        pltpu.VMEM((CHUNK, D), jnp.float32),
        pltpu.VMEM_SHARED((N_PAD, D), jnp.float32),
        pltpu.SemaphoreType.DMA,
        pltpu.SemaphoreType.DMA,
        pltpu.SemaphoreType.DMA,
        pltpu.SemaphoreType.DMA,
        pltpu.SemaphoreType.DMA,
        pltpu.SemaphoreType.DMA,
    ],
)


@jax.jit
def kernel(x, edge_index, W_mu, b_mu, W_logstd, b_logstd):
    src = edge_index[0].astype(jnp.int32)
    dst = edge_index[1].astype(jnp.int32)
    pad = jnp.full((E_PAD - N_EDGES,), N_PAD - 1, jnp.int32)
    src2 = jnp.concatenate([src, pad]).reshape(N_CHUNKS, CHUNK)
    dst2 = jnp.concatenate([dst, pad]).reshape(N_CHUNKS, CHUNK)

    cnt = _deg_call(dst2)

    y, s = pl.pallas_call(
        _scale_tc_body,
        out_shape=(
            jax.ShapeDtypeStruct((N_PAD, D), jnp.float32),
            jax.ShapeDtypeStruct((N_NODES, 1), jnp.float32),
        ),
    )(x, cnt)

    z = _scatter_call(y, src2, dst2)

    mu, logstd = pl.pallas_call(
        _matmul_tc_body,
        out_shape=(
            jax.ShapeDtypeStruct((N_NODES, D), jnp.float32),
            jax.ShapeDtypeStruct((N_NODES, D), jnp.float32),
        ),
    )(z, y, s, W_mu, b_mu.reshape(1, D), W_logstd, b_logstd.reshape(1, D))

    return (mu, logstd)


# speedup vs baseline: 1.1055x; 1.1055x over previous
import jax
import jax.numpy as jnp
from jax import lax
from jax.experimental import pallas as pl
from jax.experimental.pallas import tpu as pltpu
from jax.experimental.pallas import tpu_sc as plsc

N_NODES = 10000
N_PAD = 10240
D = 128
N_EDGES = 320000

NC = 2
NS = 16
NW = NC * NS
CHUNK = 128
N_CHUNKS = 2560
STEPS0 = 128
STEPS1 = (N_CHUNKS - NS * STEPS0) // NS
E_PAD = N_CHUNKS * CHUNK
DEG_STEPS = N_CHUNKS // NW
ROWS_PER_TILE = N_PAD // NS
DEG_W = 16
IB = 8
DEG_GRP = 8


def _deg_sc_body(dst_hbm, cnt_hbm, didx_all, ones_v, zbuf, deg_sh, dsem):
    c = lax.axis_index("c")
    s = lax.axis_index("s")
    wid = c * NS + s
    rlo = s * ROWS_PER_TILE

    one16 = jnp.full((16,), 1.0, dtype=jnp.float32)
    zero16 = jnp.zeros((16,), dtype=jnp.float32)
    def fill(i, _):
        ones_v[i, pl.ds(0, 16)] = one16
        zbuf[i, pl.ds(0, 16)] = zero16
        return 0
    lax.fori_loop(0, CHUNK, fill, 0)

    pltpu.sync_copy(dst_hbm.at[pl.ds(wid * DEG_STEPS, DEG_STEPS)], didx_all)
    for k in range(ROWS_PER_TILE // CHUNK):
        pltpu.sync_copy(zbuf, deg_sh.at[pl.ds(rlo + k * CHUNK, CHUNK)])
    plsc.subcore_barrier()

    def group(g, _):
        for j in range(DEG_GRP):
            pltpu.async_copy(ones_v, deg_sh.at[didx_all.at[g * DEG_GRP + j]],
                             dsem, add=True)
        for j in range(DEG_GRP):
            pltpu.make_async_copy(ones_v, deg_sh.at[didx_all.at[0]],
                                  dsem).wait()
        return 0
    lax.fori_loop(0, DEG_STEPS // DEG_GRP, group, 0)

    plsc.subcore_barrier()
    pltpu.sync_copy(deg_sh.at[pl.ds(rlo, ROWS_PER_TILE)],
                    cnt_hbm.at[c, pl.ds(rlo, ROWS_PER_TILE)])


def _scatter_sc_body(y_hbm, src_hbm, dst_hbm, z_hbm,
                     sidx3, didx3, rb0, rb1, z_sh,
                     isem0, isem1, gsem0, gsem1, ssem0, ssem1):
    c = lax.axis_index("c")
    s = lax.axis_index("s")
    rlo = s * ROWS_PER_TILE
    rows = [rb0, rb1]
    isem = [isem0, isem1]
    gsem = [gsem0, gsem1]
    ssem = [ssem0, ssem1]
    cbase = jnp.where(c == 0, s * STEPS0, NS * STEPS0 + s * STEPS1)
    ng = jnp.where(c == 0, STEPS0 // IB, STEPS1 // IB)

    zero16 = jnp.zeros((16,), dtype=jnp.float32)
    def fill(i, _):
        for j in range(D // 16):
            rb0[i, pl.ds(j * 16, 16)] = zero16
        return 0
    lax.fori_loop(0, CHUNK, fill, 0)
    for k in range(ROWS_PER_TILE // CHUNK):
        pltpu.sync_copy(rb0, z_sh.at[pl.ds(rlo + k * CHUNK, CHUNK)])
    plsc.subcore_barrier()

    def iload(gi, p):
        pltpu.async_copy(src_hbm.at[pl.ds(cbase + gi * IB, IB)],
                         sidx3.at[p], isem[p])
        pltpu.async_copy(dst_hbm.at[pl.ds(cbase + gi * IB, IB)],
                         didx3.at[p], isem[p])
    def iwait(p):
        pltpu.make_async_copy(src_hbm.at[pl.ds(cbase, IB)], sidx3.at[p],
                              isem[p]).wait()
        pltpu.make_async_copy(dst_hbm.at[pl.ds(cbase, IB)], didx3.at[p],
                              isem[p]).wait()
    def gstart(p, k, b):
        pltpu.async_copy(y_hbm.at[sidx3.at[p, k]], rows[b], gsem[b])
    def gwait(b):
        pltpu.make_async_copy(y_hbm.at[sidx3.at[0, 0]], rows[b],
                              gsem[b]).wait()
    def sstart(p, k, b):
        pltpu.async_copy(rows[b], z_sh.at[didx3.at[p, k]], ssem[b], add=True)
    def swait(b):
        pltpu.make_async_copy(rows[b], z_sh.at[didx3.at[0, 0]],
                              ssem[b]).wait()

    def group(gi, p):
        iwait(p)
        d = pltpu.async_copy(y_hbm.at[sidx3.at[p, 0]], rows[0], gsem[0])
        sd = [None, None]
        for k in range(IB):
            b = k % 2
            d.wait()
            if sd[b] is not None:
                sd[b].wait()
            sd[b] = pltpu.async_copy(rows[b], z_sh.at[didx3.at[p, k]],
                                     ssem[b], add=True)
            if k + 1 < IB:
                if sd[1 - b] is not None:
                    sd[1 - b].wait()
                    sd[1 - b] = None
                d = pltpu.async_copy(y_hbm.at[sidx3.at[p, k + 1]],
                                     rows[1 - b], gsem[1 - b])
        sd[0].wait()
        sd[1].wait()
        @pl.when(gi < ng - 2)
        def _():
            iload(gi + 2, p)

    iload(0, 0)
    iload(1, 1)
    def outer(t, _):
        group(2 * t, 0)
        group(2 * t + 1, 1)
        return 0
    lax.fori_loop(0, ng // 2, outer, 0)

    plsc.subcore_barrier()
    pltpu.sync_copy(z_sh.at[pl.ds(rlo, ROWS_PER_TILE)],
                    z_hbm.at[c, pl.ds(rlo, ROWS_PER_TILE)])


def _scale_tc_body(x_ref, cnt_ref, y_ref, s_ref):
    cnt = cnt_ref[0, 0:N_NODES, 0:1] + cnt_ref[1, 0:N_NODES, 0:1]
    s = lax.rsqrt(cnt + 1.0)
    s_ref[...] = s
    y_ref[0:N_NODES, :] = x_ref[...] * s
    y_ref[N_NODES:N_PAD, :] = jnp.zeros((N_PAD - N_NODES, D), jnp.float32)


def _matmul_tc_body(z_ref, y_ref, s_ref, wm_ref, bm_ref, wl_ref, bl_ref,
                    mu_ref, ls_ref):
    agg = (z_ref[0, 0:N_NODES, :] + z_ref[1, 0:N_NODES, :]
           + y_ref[0:N_NODES, :]) * s_ref[...]
    mu_ref[...] = jnp.dot(agg, wm_ref[...],
                          preferred_element_type=jnp.float32,
                          precision=lax.Precision.HIGHEST) + bm_ref[...]
    ls_ref[...] = jnp.dot(agg, wl_ref[...],
                          preferred_element_type=jnp.float32,
                          precision=lax.Precision.HIGHEST) + bl_ref[...]


_SC_MESH = plsc.VectorSubcoreMesh(core_axis_name="c", subcore_axis_name="s")

_deg_call = pl.kernel(
    _deg_sc_body,
    out_type=jax.ShapeDtypeStruct((NC, N_PAD, DEG_W), jnp.float32),
    mesh=_SC_MESH,
    scratch_types=[
        pltpu.VMEM((DEG_STEPS, CHUNK), jnp.int32),
        pltpu.VMEM((CHUNK, DEG_W), jnp.float32),
        pltpu.VMEM((CHUNK, DEG_W), jnp.float32),
        pltpu.VMEM_SHARED((N_PAD, DEG_W), jnp.float32),
        pltpu.SemaphoreType.DMA,
    ],
)

_scatter_call = pl.kernel(
    _scatter_sc_body,
    out_type=jax.ShapeDtypeStruct((NC, N_PAD, D), jnp.float32),
    mesh=_SC_MESH,
    scratch_types=[
        pltpu.VMEM((2, IB, CHUNK), jnp.int32),
        pltpu.VMEM((2, IB, CHUNK), jnp.int32),
        pltpu.VMEM((CHUNK, D), jnp.float32),
        pltpu.VMEM((CHUNK, D), jnp.float32),
        pltpu.VMEM_SHARED((N_PAD, D), jnp.float32),
        pltpu.SemaphoreType.DMA,
        pltpu.SemaphoreType.DMA,
        pltpu.SemaphoreType.DMA,
        pltpu.SemaphoreType.DMA,
        pltpu.SemaphoreType.DMA,
        pltpu.SemaphoreType.DMA,
    ],
)


@jax.jit
def kernel(x, edge_index, W_mu, b_mu, W_logstd, b_logstd):
    src = edge_index[0].astype(jnp.int32)
    dst = edge_index[1].astype(jnp.int32)
    pad = jnp.full((E_PAD - N_EDGES,), N_PAD - 1, jnp.int32)
    src2 = jnp.concatenate([src, pad]).reshape(N_CHUNKS, CHUNK)
    dst2 = jnp.concatenate([dst, pad]).reshape(N_CHUNKS, CHUNK)

    cnt = _deg_call(dst2)

    y, s = pl.pallas_call(
        _scale_tc_body,
        out_shape=(
            jax.ShapeDtypeStruct((N_PAD, D), jnp.float32),
            jax.ShapeDtypeStruct((N_NODES, 1), jnp.float32),
        ),
    )(x, cnt)

    z = _scatter_call(y, src2, dst2)

    mu, logstd = pl.pallas_call(
        _matmul_tc_body,
        out_shape=(
            jax.ShapeDtypeStruct((N_NODES, D), jnp.float32),
            jax.ShapeDtypeStruct((N_NODES, D), jnp.float32),
        ),
    )(z, y, s, W_mu, b_mu.reshape(1, D), W_logstd, b_logstd.reshape(1, D))

    return (mu, logstd)


# speedup vs baseline: 1.1055x; 1.0000x over previous
import jax
import jax.numpy as jnp
from jax import lax
from jax.experimental import pallas as pl
from jax.experimental.pallas import tpu as pltpu
from jax.experimental.pallas import tpu_sc as plsc

N_NODES = 10000
N_PAD = 10240
D = 128
N_EDGES = 320000

NC = 2
NS = 16
NW = NC * NS
CHUNK = 128
N_CHUNKS = 2560
STEPS0 = 128
STEPS1 = (N_CHUNKS - NS * STEPS0) // NS
E_PAD = N_CHUNKS * CHUNK
DEG_STEPS = N_CHUNKS // NW
ROWS_PER_TILE = N_PAD // NS
DEG_W = 16
IB = 8
DEG_GRP = 8


def _deg_sc_body(dst_hbm, cnt_hbm, didx_all, ones_v, zbuf, deg_sh, dsem):
    c = lax.axis_index("c")
    s = lax.axis_index("s")
    wid = c * NS + s
    rlo = s * ROWS_PER_TILE

    one16 = jnp.full((16,), 1.0, dtype=jnp.float32)
    zero16 = jnp.zeros((16,), dtype=jnp.float32)
    def fill(i, _):
        ones_v[i, pl.ds(0, 16)] = one16
        zbuf[i, pl.ds(0, 16)] = zero16
        return 0
    lax.fori_loop(0, CHUNK, fill, 0)

    pltpu.sync_copy(dst_hbm.at[pl.ds(wid * DEG_STEPS, DEG_STEPS)], didx_all)
    for k in range(ROWS_PER_TILE // CHUNK):
        pltpu.sync_copy(zbuf, deg_sh.at[pl.ds(rlo + k * CHUNK, CHUNK)])
    plsc.subcore_barrier()

    def group(g, _):
        for j in range(DEG_GRP):
            pltpu.async_copy(ones_v, deg_sh.at[didx_all.at[g * DEG_GRP + j]],
                             dsem, add=True)
        for j in range(DEG_GRP):
            pltpu.make_async_copy(ones_v, deg_sh.at[didx_all.at[0]],
                                  dsem).wait()
        return 0
    lax.fori_loop(0, DEG_STEPS // DEG_GRP, group, 0)

    plsc.subcore_barrier()
    pltpu.sync_copy(deg_sh.at[pl.ds(rlo, ROWS_PER_TILE)],
                    cnt_hbm.at[c, pl.ds(rlo, ROWS_PER_TILE)])


def _scatter_sc_body(y_hbm, src_hbm, dst_hbm, z_hbm,
                     sidx3, didx3, rb0, rb1, z_sh,
                     isem0, isem1, gsem0, gsem1, ssem0, ssem1):
    c = lax.axis_index("c")
    s = lax.axis_index("s")
    rlo = s * ROWS_PER_TILE
    rows = [rb0, rb1]
    isem = [isem0, isem1]
    gsem = [gsem0, gsem1]
    ssem = [ssem0, ssem1]

    zero16 = jnp.zeros((16,), dtype=jnp.float32)
    def fill(i, _):
        for j in range(D // 16):
            rb0[i, pl.ds(j * 16, 16)] = zero16
        return 0
    lax.fori_loop(0, CHUNK, fill, 0)
    for k in range(ROWS_PER_TILE // CHUNK):
        pltpu.sync_copy(rb0, z_sh.at[pl.ds(rlo + k * CHUNK, CHUNK)])
    plsc.subcore_barrier()

    def iwait(p):
        pltpu.make_async_copy(src_hbm.at[pl.ds(0, IB)], sidx3.at[p],
                              isem[p]).wait()
        pltpu.make_async_copy(dst_hbm.at[pl.ds(0, IB)], didx3.at[p],
                              isem[p]).wait()
    def gstart(p, k, b):
        pltpu.async_copy(y_hbm.at[sidx3.at[p, k]], rows[b], gsem[b])
    def gwait(b):
        pltpu.make_async_copy(y_hbm.at[sidx3.at[0, 0]], rows[b],
                              gsem[b]).wait()
    def sstart(p, k, b):
        pltpu.async_copy(rows[b], z_sh.at[didx3.at[p, k]], ssem[b], add=True)
    def swait(b):
        pltpu.make_async_copy(rows[b], z_sh.at[didx3.at[0, 0]],
                              ssem[b]).wait()

    def pipeline(cbase, n_groups):
        def iload(gi, p):
            pltpu.async_copy(src_hbm.at[pl.ds(cbase + gi * IB, IB)],
                             sidx3.at[p], isem[p])
            pltpu.async_copy(dst_hbm.at[pl.ds(cbase + gi * IB, IB)],
                             didx3.at[p], isem[p])

        def group(gi, p):
            iwait(p)
            d = pltpu.async_copy(y_hbm.at[sidx3.at[p, 0]], rows[0], gsem[0])
            sd = [None, None]
            for k in range(IB):
                b = k % 2
                d.wait()
                sd[b] = pltpu.async_copy(rows[b], z_sh.at[didx3.at[p, k]],
                                         ssem[b], add=True)
                if k + 1 < IB:
                    if sd[1 - b] is not None:
                        sd[1 - b].wait()
                        sd[1 - b] = None
                    d = pltpu.async_copy(y_hbm.at[sidx3.at[p, k + 1]],
                                         rows[1 - b], gsem[1 - b])
            sd[0].wait()
            sd[1].wait()
            @pl.when(gi < n_groups - 2)
            def _():
                iload(gi + 2, p)

        iload(0, 0)
        iload(1, 1)
        def outer(t, _):
            group(2 * t, 0)
            group(2 * t + 1, 1)
            return 0
        lax.fori_loop(0, n_groups // 2, outer, 0)

    @pl.when(c == 0)
    def _():
        pipeline(s * STEPS0, STEPS0 // IB)
    @pl.when(c != 0)
    def _():
        pipeline(NS * STEPS0 + s * STEPS1, STEPS1 // IB)

    plsc.subcore_barrier()
    pltpu.sync_copy(z_sh.at[pl.ds(rlo, ROWS_PER_TILE)],
                    z_hbm.at[c, pl.ds(rlo, ROWS_PER_TILE)])


def _scale_tc_body(x_ref, cnt_ref, y_ref, s_ref):
    cnt = cnt_ref[0, 0:N_NODES, 0:1] + cnt_ref[1, 0:N_NODES, 0:1]
    s = lax.rsqrt(cnt + 1.0)
    s_ref[...] = s
    y_ref[0:N_NODES, :] = x_ref[...] * s
    y_ref[N_NODES:N_PAD, :] = jnp.zeros((N_PAD - N_NODES, D), jnp.float32)


def _matmul_tc_body(z_ref, y_ref, s_ref, wm_ref, bm_ref, wl_ref, bl_ref,
                    mu_ref, ls_ref):
    agg = (z_ref[0, 0:N_NODES, :] + z_ref[1, 0:N_NODES, :]
           + y_ref[0:N_NODES, :]) * s_ref[...]
    mu_ref[...] = jnp.dot(agg, wm_ref[...],
                          preferred_element_type=jnp.float32,
                          precision=lax.Precision.HIGHEST) + bm_ref[...]
    ls_ref[...] = jnp.dot(agg, wl_ref[...],
                          preferred_element_type=jnp.float32,
                          precision=lax.Precision.HIGHEST) + bl_ref[...]


_SC_MESH = plsc.VectorSubcoreMesh(core_axis_name="c", subcore_axis_name="s")

_deg_call = pl.kernel(
    _deg_sc_body,
    out_type=jax.ShapeDtypeStruct((NC, N_PAD, DEG_W), jnp.float32),
    mesh=_SC_MESH,
    scratch_types=[
        pltpu.VMEM((DEG_STEPS, CHUNK), jnp.int32),
        pltpu.VMEM((CHUNK, DEG_W), jnp.float32),
        pltpu.VMEM((CHUNK, DEG_W), jnp.float32),
        pltpu.VMEM_SHARED((N_PAD, DEG_W), jnp.float32),
        pltpu.SemaphoreType.DMA,
    ],
)

_scatter_call = pl.kernel(
    _scatter_sc_body,
    out_type=jax.ShapeDtypeStruct((NC, N_PAD, D), jnp.float32),
    mesh=_SC_MESH,
    scratch_types=[
        pltpu.VMEM((2, IB, CHUNK), jnp.int32),
        pltpu.VMEM((2, IB, CHUNK), jnp.int32),
        pltpu.VMEM((CHUNK, D), jnp.float32),
        pltpu.VMEM((CHUNK, D), jnp.float32),
        pltpu.VMEM_SHARED((N_PAD, D), jnp.float32),
        pltpu.SemaphoreType.DMA,
        pltpu.SemaphoreType.DMA,
        pltpu.SemaphoreType.DMA,
        pltpu.SemaphoreType.DMA,
        pltpu.SemaphoreType.DMA,
        pltpu.SemaphoreType.DMA,
    ],
)


@jax.jit
def kernel(x, edge_index, W_mu, b_mu, W_logstd, b_logstd):
    src = edge_index[0].astype(jnp.int32)
    dst = edge_index[1].astype(jnp.int32)
    pad = jnp.full((E_PAD - N_EDGES,), N_PAD - 1, jnp.int32)
    src2 = jnp.concatenate([src, pad]).reshape(N_CHUNKS, CHUNK)
    dst2 = jnp.concatenate([dst, pad]).reshape(N_CHUNKS, CHUNK)

    cnt = _deg_call(dst2)

    y, s = pl.pallas_call(
        _scale_tc_body,
        out_shape=(
            jax.ShapeDtypeStruct((N_PAD, D), jnp.float32),
            jax.ShapeDtypeStruct((N_NODES, 1), jnp.float32),
        ),
    )(x, cnt)

    z = _scatter_call(y, src2, dst2)

    mu, logstd = pl.pallas_call(
        _matmul_tc_body,
        out_shape=(
            jax.ShapeDtypeStruct((N_NODES, D), jnp.float32),
            jax.ShapeDtypeStruct((N_NODES, D), jnp.float32),
        ),
    )(z, y, s, W_mu, b_mu.reshape(1, D), W_logstd, b_logstd.reshape(1, D))

    return (mu, logstd)


# speedup vs baseline: 3.0788x; 2.7850x over previous
import jax
import jax.numpy as jnp
from jax import lax
from jax.experimental import pallas as pl
from jax.experimental.pallas import tpu as pltpu
from jax.experimental.pallas import tpu_sc as plsc

N_NODES = 10000
N_PAD = 10240
D = 128
N_EDGES = 320000

NC = 2
NS = 16
NW = NC * NS
CHUNK = 128
N_CHUNKS = 2560
STEPS = N_CHUNKS // NW
E_PAD = N_CHUNKS * CHUNK
DEG_STEPS = N_CHUNKS // NW
ROWS_PER_TILE = N_PAD // NS
DEG_W = 16
IB = 8
DEG_GRP = 8


def _deg_sc_body(dst_hbm, cnt_hbm, didx_all, ones_v, zbuf, deg_sh, dsem):
    c = lax.axis_index("c")
    s = lax.axis_index("s")
    wid = c * NS + s
    rlo = s * ROWS_PER_TILE

    one16 = jnp.full((16,), 1.0, dtype=jnp.float32)
    zero16 = jnp.zeros((16,), dtype=jnp.float32)
    def fill(i, _):
        ones_v[i, pl.ds(0, 16)] = one16
        zbuf[i, pl.ds(0, 16)] = zero16
        return 0
    lax.fori_loop(0, CHUNK, fill, 0)

    pltpu.sync_copy(dst_hbm.at[pl.ds(wid * DEG_STEPS, DEG_STEPS)], didx_all)
    for k in range(ROWS_PER_TILE // CHUNK):
        pltpu.sync_copy(zbuf, deg_sh.at[pl.ds(rlo + k * CHUNK, CHUNK)])
    plsc.subcore_barrier()

    def group(g, _):
        for j in range(DEG_GRP):
            pltpu.async_copy(ones_v, deg_sh.at[didx_all.at[g * DEG_GRP + j]],
                             dsem, add=True)
        for j in range(DEG_GRP):
            pltpu.make_async_copy(ones_v, deg_sh.at[didx_all.at[0]],
                                  dsem).wait()
        return 0
    lax.fori_loop(0, DEG_STEPS // DEG_GRP, group, 0)

    plsc.subcore_barrier()
    pltpu.sync_copy(deg_sh.at[pl.ds(rlo, ROWS_PER_TILE)],
                    cnt_hbm.at[c, pl.ds(rlo, ROWS_PER_TILE)])


def _scatter_sc_body(y_hbm, src_hbm, dst_hbm, z_hbm,
                     sidx3, didx3, rb0, rb1, z_sh,
                     isem0, isem1, gsem0, gsem1, ssem0, ssem1):
    c = lax.axis_index("c")
    s = lax.axis_index("s")
    rlo = s * ROWS_PER_TILE
    rows = [rb0, rb1]
    isem = [isem0, isem1]
    gsem = [gsem0, gsem1]
    ssem = [ssem0, ssem1]

    zero16 = jnp.zeros((16,), dtype=jnp.float32)
    def fill(i, _):
        for j in range(D // 16):
            rb0[i, pl.ds(j * 16, 16)] = zero16
        return 0
    lax.fori_loop(0, CHUNK, fill, 0)
    for k in range(ROWS_PER_TILE // CHUNK):
        pltpu.sync_copy(rb0, z_sh.at[pl.ds(rlo + k * CHUNK, CHUNK)])
    plsc.subcore_barrier()

    def iwait(p):
        pltpu.make_async_copy(src_hbm.at[pl.ds(0, IB)], sidx3.at[p],
                              isem[p]).wait()
        pltpu.make_async_copy(dst_hbm.at[pl.ds(0, IB)], didx3.at[p],
                              isem[p]).wait()
    def gstart(p, k, b):
        pltpu.async_copy(y_hbm.at[sidx3.at[p, k]], rows[b], gsem[b])
    def gwait(b):
        pltpu.make_async_copy(y_hbm.at[sidx3.at[0, 0]], rows[b],
                              gsem[b]).wait()
    def sstart(p, k, b):
        pltpu.async_copy(rows[b], z_sh.at[didx3.at[p, k]], ssem[b], add=True)
    def swait(b):
        pltpu.make_async_copy(rows[b], z_sh.at[didx3.at[0, 0]],
                              ssem[b]).wait()

    def pipeline(cbase, n_groups):
        def iload(gi, p):
            pltpu.async_copy(src_hbm.at[pl.ds(cbase + gi * IB, IB)],
                             sidx3.at[p], isem[p])
            pltpu.async_copy(dst_hbm.at[pl.ds(cbase + gi * IB, IB)],
                             didx3.at[p], isem[p])

        def group(gi, p):
            iwait(p)
            d = pltpu.async_copy(y_hbm.at[sidx3.at[p, 0]], rows[0], gsem[0])
            sd = [None, None]
            for k in range(IB):
                b = k % 2
                d.wait()
                sd[b] = pltpu.async_copy(rows[b], z_sh.at[didx3.at[p, k]],
                                         ssem[b], add=True)
                if k + 1 < IB:
                    if sd[1 - b] is not None:
                        sd[1 - b].wait()
                        sd[1 - b] = None
                    d = pltpu.async_copy(y_hbm.at[sidx3.at[p, k + 1]],
                                         rows[1 - b], gsem[1 - b])
            sd[0].wait()
            sd[1].wait()
            @pl.when(gi < n_groups - 2)
            def _():
                iload(gi + 2, p)

        iload(0, 0)
        iload(1, 1)
        def outer(t, _):
            group(2 * t, 0)
            group(2 * t + 1, 1)
            return 0
        lax.fori_loop(0, n_groups // 2, outer, 0)

    pipeline((c * NS + s) * STEPS, STEPS // IB)

    plsc.subcore_barrier()
    pltpu.sync_copy(z_sh.at[pl.ds(rlo, ROWS_PER_TILE)],
                    z_hbm.at[c, pl.ds(rlo, ROWS_PER_TILE)])


def _scale_tc_body(x_ref, cnt_ref, y_ref, s_ref):
    cnt = cnt_ref[0, 0:N_NODES, 0:1] + cnt_ref[1, 0:N_NODES, 0:1]
    s = lax.rsqrt(cnt + 1.0)
    s_ref[...] = s
    y_ref[0:N_NODES, :] = x_ref[...] * s
    y_ref[N_NODES:N_PAD, :] = jnp.zeros((N_PAD - N_NODES, D), jnp.float32)


def _matmul_tc_body(z_ref, y_ref, s_ref, wm_ref, bm_ref, wl_ref, bl_ref,
                    mu_ref, ls_ref):
    agg = (z_ref[0, 0:N_NODES, :] + z_ref[1, 0:N_NODES, :]
           + y_ref[0:N_NODES, :]) * s_ref[...]
    mu_ref[...] = jnp.dot(agg, wm_ref[...],
                          preferred_element_type=jnp.float32,
                          precision=lax.Precision.HIGHEST) + bm_ref[...]
    ls_ref[...] = jnp.dot(agg, wl_ref[...],
                          preferred_element_type=jnp.float32,
                          precision=lax.Precision.HIGHEST) + bl_ref[...]


_SC_MESH = plsc.VectorSubcoreMesh(core_axis_name="c", subcore_axis_name="s")

_deg_call = pl.kernel(
    _deg_sc_body,
    out_type=jax.ShapeDtypeStruct((NC, N_PAD, DEG_W), jnp.float32),
    mesh=_SC_MESH,
    scratch_types=[
        pltpu.VMEM((DEG_STEPS, CHUNK), jnp.int32),
        pltpu.VMEM((CHUNK, DEG_W), jnp.float32),
        pltpu.VMEM((CHUNK, DEG_W), jnp.float32),
        pltpu.VMEM_SHARED((N_PAD, DEG_W), jnp.float32),
        pltpu.SemaphoreType.DMA,
    ],
)

_scatter_call = pl.kernel(
    _scatter_sc_body,
    out_type=jax.ShapeDtypeStruct((NC, N_PAD, D), jnp.float32),
    mesh=_SC_MESH,
    scratch_types=[
        pltpu.VMEM((2, IB, CHUNK), jnp.int32),
        pltpu.VMEM((2, IB, CHUNK), jnp.int32),
        pltpu.VMEM((CHUNK, D), jnp.float32),
        pltpu.VMEM((CHUNK, D), jnp.float32),
        pltpu.VMEM_SHARED((N_PAD, D), jnp.float32),
        pltpu.SemaphoreType.DMA,
        pltpu.SemaphoreType.DMA,
        pltpu.SemaphoreType.DMA,
        pltpu.SemaphoreType.DMA,
        pltpu.SemaphoreType.DMA,
        pltpu.SemaphoreType.DMA,
    ],
)


@jax.jit
def kernel(x, edge_index, W_mu, b_mu, W_logstd, b_logstd):
    src = edge_index[0].astype(jnp.int32)
    dst = edge_index[1].astype(jnp.int32)
    pad = (jnp.arange(E_PAD - N_EDGES, dtype=jnp.int32)
           % (N_PAD - N_NODES)) + N_NODES
    src2 = jnp.concatenate([src, pad]).reshape(N_CHUNKS, CHUNK)
    dst2 = jnp.concatenate([dst, pad]).reshape(N_CHUNKS, CHUNK)

    cnt = _deg_call(dst2)

    y, s = pl.pallas_call(
        _scale_tc_body,
        out_shape=(
            jax.ShapeDtypeStruct((N_PAD, D), jnp.float32),
            jax.ShapeDtypeStruct((N_NODES, 1), jnp.float32),
        ),
    )(x, cnt)

    z = _scatter_call(y, src2, dst2)

    mu, logstd = pl.pallas_call(
        _matmul_tc_body,
        out_shape=(
            jax.ShapeDtypeStruct((N_NODES, D), jnp.float32),
            jax.ShapeDtypeStruct((N_NODES, D), jnp.float32),
        ),
    )(z, y, s, W_mu, b_mu.reshape(1, D), W_logstd, b_logstd.reshape(1, D))

    return (mu, logstd)
